# Initial kernel scaffold; baseline (speedup 1.0000x reference)
import jax
import jax.numpy as jnp
from jax.experimental import pallas as pl


def kernel(dayofweek, time, sex, age, month, day, W_dayofweek, W_time, W_sex, W_age, W_month, W_day):
    raise NotImplementedError("write your pallas kernel here")


# speedup vs baseline: 1.4762x; 1.4762x over previous
import functools

import jax
import jax.numpy as jnp
from jax import lax
from jax.experimental import pallas as pl
from jax.experimental.pallas import tpu as pltpu
from jax.experimental.pallas import tpu_sc as plsc

NUM_DIM = 64
BATCH = 16384
NUM_FEATURES = 6
_VOCABS = (7, 24, 2, 100, 12, 31)
TOTAL_ROWS = BATCH * NUM_FEATURES

_NC = 2
_NS = 16
_NW = _NC * _NS
_R_PER_W = TOTAL_ROWS // _NW
_CHUNK = 128
_BLOCK = 512
_CPB = _BLOCK // _CHUNK
_NBLOCK = _R_PER_W // _BLOCK
_IDX_ROWS = TOTAL_ROWS // _CHUNK
_IDXR_PER_W = _IDX_ROWS // _NW

_mesh = plsc.VectorSubcoreMesh(core_axis_name="c", subcore_axis_name="s")


@functools.partial(
    pl.kernel,
    out_type=jax.ShapeDtypeStruct((TOTAL_ROWS, NUM_DIM), jnp.float32),
    mesh=_mesh,
    compiler_params=pltpu.CompilerParams(use_tc_tiling_on_sc=False),
    scratch_types=[
        pltpu.VMEM((_IDXR_PER_W, _CHUNK), jnp.int32),
        pltpu.VMEM((_BLOCK, NUM_DIM), jnp.float32),
        pltpu.VMEM((_BLOCK, NUM_DIM), jnp.float32),
        pltpu.SemaphoreType.DMA,
        pltpu.SemaphoreType.DMA,
    ],
)
def _embed_concat(idx_hbm, table_hbm, out, idx_v, rows_a, rows_b, gsem, osem):
    bufs = (rows_a, rows_b)
    wid = lax.axis_index("s") * _NC + lax.axis_index("c")
    pltpu.sync_copy(idx_hbm.at[pl.ds(wid * _IDXR_PER_W, _IDXR_PER_W)], idx_v)

    out_base = wid * _R_PER_W
    out_writes = []
    for t in range(_NBLOCK):
        buf = bufs[t % 2]
        if t >= 2:
            out_writes[t - 2].wait()
        gathers = [
            pltpu.async_copy(
                table_hbm.at[idx_v.at[t * _CPB + j]],
                buf.at[pl.ds(j * _CHUNK, _CHUNK)],
                gsem,
            )
            for j in range(_CPB)
        ]
        for g in gathers:
            g.wait()
        out_writes.append(
            pltpu.async_copy(
                buf, out.at[pl.ds(out_base + t * _BLOCK, _BLOCK)], osem
            )
        )
    out_writes[_NBLOCK - 2].wait()
    out_writes[_NBLOCK - 1].wait()


def kernel(dayofweek, time, sex, age, month, day,
           W_dayofweek, W_time, W_sex, W_age, W_month, W_day):
    tables = (W_dayofweek, W_time, W_sex, W_age, W_month, W_day)
    table_all = jnp.concatenate(tables, axis=0)
    offsets = []
    off = 0
    for v in _VOCABS:
        offsets.append(off)
        off += v
    idx_all = jnp.stack(
        [a.astype(jnp.int32) + o
         for a, o in zip((dayofweek, time, sex, age, month, day), offsets)],
        axis=-1,
    ).reshape(_IDX_ROWS, _CHUNK)
    out = _embed_concat(idx_all, table_all)
    return out.reshape(BATCH, NUM_FEATURES * NUM_DIM)


# speedup vs baseline: 5.0022x; 3.3887x over previous
import functools

import jax
import jax.numpy as jnp
from jax import lax
from jax.experimental import pallas as pl
from jax.experimental.pallas import tpu as pltpu
from jax.experimental.pallas import tpu_sc as plsc

NUM_DIM = 64
BATCH = 16384
NUM_FEATURES = 6
_VOCABS = (7, 24, 2, 100, 12, 31)
TOTAL_ROWS = BATCH * NUM_FEATURES

_NC = 2
_NS = 16
_NW = _NC * _NS
_R_PER_W = TOTAL_ROWS // _NW
_CHUNK = 128
_BLOCK = 512
_CPB = _BLOCK // _CHUNK
_NBLOCK = _R_PER_W // _BLOCK
_IDX_ROWS = TOTAL_ROWS // _CHUNK
_IDXR_PER_W = _IDX_ROWS // _NW

_mesh = plsc.VectorSubcoreMesh(core_axis_name="c", subcore_axis_name="s")


@functools.partial(
    pl.kernel,
    out_type=jax.ShapeDtypeStruct((TOTAL_ROWS, NUM_DIM), jnp.float32),
    mesh=_mesh,
    compiler_params=pltpu.CompilerParams(use_tc_tiling_on_sc=False),
    scratch_types=[
        pltpu.VMEM((_IDXR_PER_W, _CHUNK), jnp.int32),
        pltpu.VMEM((_BLOCK, NUM_DIM), jnp.float32),
        pltpu.VMEM((_BLOCK, NUM_DIM), jnp.float32),
        pltpu.VMEM_SHARED((176, NUM_DIM), jnp.float32),
        pltpu.SemaphoreType.DMA,
        pltpu.SemaphoreType.DMA,
    ],
)
def _embed_concat(idx_hbm, table_hbm, out, idx_v, rows_a, rows_b, table_sh,
                  gsem, osem):
    bufs = (rows_a, rows_b)
    sid = lax.axis_index("s")
    wid = sid * _NC + lax.axis_index("c")

    @pl.when(sid == 0)
    def _():
        pltpu.sync_copy(table_hbm, table_sh)

    pltpu.sync_copy(idx_hbm.at[pl.ds(wid * _IDXR_PER_W, _IDXR_PER_W)], idx_v)
    plsc.subcore_barrier()

    out_base = wid * _R_PER_W
    out_writes = []
    for t in range(_NBLOCK):
        buf = bufs[t % 2]
        if t >= 2:
            out_writes[t - 2].wait()
        gathers = [
            pltpu.async_copy(
                table_sh.at[idx_v.at[t * _CPB + j]],
                buf.at[pl.ds(j * _CHUNK, _CHUNK)],
                gsem,
            )
            for j in range(_CPB)
        ]
        for g in gathers:
            g.wait()
        out_writes.append(
            pltpu.async_copy(
                buf, out.at[pl.ds(out_base + t * _BLOCK, _BLOCK)], osem
            )
        )
    out_writes[_NBLOCK - 2].wait()
    out_writes[_NBLOCK - 1].wait()


def kernel(dayofweek, time, sex, age, month, day,
           W_dayofweek, W_time, W_sex, W_age, W_month, W_day):
    tables = (W_dayofweek, W_time, W_sex, W_age, W_month, W_day)
    table_all = jnp.concatenate(tables, axis=0)
    offsets = []
    off = 0
    for v in _VOCABS:
        offsets.append(off)
        off += v
    idx_all = jnp.stack(
        [a.astype(jnp.int32) + o
         for a, o in zip((dayofweek, time, sex, age, month, day), offsets)],
        axis=-1,
    ).reshape(_IDX_ROWS, _CHUNK)
    out = _embed_concat(idx_all, table_all)
    return out.reshape(BATCH, NUM_FEATURES * NUM_DIM)


# speedup vs baseline: 6.7078x; 1.3410x over previous
import functools

import jax
import jax.numpy as jnp
from jax import lax
from jax.experimental import pallas as pl
from jax.experimental.pallas import tpu as pltpu
from jax.experimental.pallas import tpu_sc as plsc

NUM_DIM = 64
BATCH = 16384
NUM_PAIRS = 3
PAIR_DIM = 2 * NUM_DIM
TABLE_ROWS = 7 * 24 + 2 * 100 + 12 * 31

_NC = 2
_NS = 16
_NW = _NC * _NS
_S_PER_W = BATCH // _NW
_CHUNK = 128
_NBLOCK = _S_PER_W // _CHUNK
_IDX_PER_W = _S_PER_W * NUM_PAIRS

_mesh = plsc.VectorSubcoreMesh(core_axis_name="c", subcore_axis_name="s")


@functools.partial(
    pl.kernel,
    out_type=jax.ShapeDtypeStruct((BATCH, NUM_PAIRS * PAIR_DIM), jnp.float32),
    mesh=_mesh,
    scratch_types=[
        pltpu.VMEM((_IDX_PER_W,), jnp.int32),
        pltpu.VMEM((_CHUNK, NUM_PAIRS * PAIR_DIM), jnp.float32),
        pltpu.VMEM((_CHUNK, NUM_PAIRS * PAIR_DIM), jnp.float32),
        pltpu.SemaphoreType.DMA,
        pltpu.SemaphoreType.DMA,
    ],
)
def _embed_concat(idx_hbm, table_hbm, out, idx_v, buf_a, buf_b, gsem, osem):
    bufs = (buf_a, buf_b)
    sid = lax.axis_index("s")
    wid = sid * _NC + lax.axis_index("c")

    pltpu.sync_copy(idx_hbm.at[pl.ds(wid * _IDX_PER_W, _IDX_PER_W)], idx_v)

    out_writes = []
    for b in range(_NBLOCK):
        buf = bufs[b % 2]
        if b >= 2:
            out_writes[b - 2].wait()
        gathers = [
            pltpu.async_copy(
                table_hbm.at[idx_v.at[pl.ds((b * NUM_PAIRS + c) * _CHUNK,
                                           _CHUNK)]],
                buf.at[:, pl.ds(c * PAIR_DIM, PAIR_DIM)],
                gsem,
            )
            for c in range(NUM_PAIRS)
        ]
        for g in gathers:
            g.wait()
        out_writes.append(
            pltpu.async_copy(
                buf,
                out.at[pl.ds(wid * _S_PER_W + b * _CHUNK, _CHUNK)],
                osem,
            )
        )
    out_writes[_NBLOCK - 2].wait()
    out_writes[_NBLOCK - 1].wait()


def _pair_table(wa, wb):
    va, vb = wa.shape[0], wb.shape[0]
    return jnp.concatenate(
        [jnp.broadcast_to(wa[:, None, :], (va, vb, NUM_DIM)),
         jnp.broadcast_to(wb[None, :, :], (va, vb, NUM_DIM))],
        axis=-1,
    ).reshape(va * vb, PAIR_DIM)


def kernel(dayofweek, time, sex, age, month, day,
           W_dayofweek, W_time, W_sex, W_age, W_month, W_day):
    table = jnp.concatenate(
        [_pair_table(W_dayofweek, W_time),
         _pair_table(W_sex, W_age),
         _pair_table(W_month, W_day)],
        axis=0,
    )
    p0 = dayofweek.astype(jnp.int32) * 24 + time.astype(jnp.int32)
    p1 = 168 + sex.astype(jnp.int32) * 100 + age.astype(jnp.int32)
    p2 = 368 + month.astype(jnp.int32) * 31 + day.astype(jnp.int32)
    idx = (jnp.stack([p0, p1, p2], axis=-1)
           .reshape(BATCH // _CHUNK, _CHUNK, NUM_PAIRS)
           .transpose(0, 2, 1)
           .reshape(-1))
    return _embed_concat(idx, table)


# speedup vs baseline: 7.9928x; 1.1916x over previous
import functools

import jax
import jax.numpy as jnp
from jax import lax
from jax.experimental import pallas as pl
from jax.experimental.pallas import tpu as pltpu
from jax.experimental.pallas import tpu_sc as plsc

NUM_DIM = 64
BATCH = 16384
NUM_PAIRS = 3
PAIR_DIM = 2 * NUM_DIM
TABLE_ROWS = 7 * 24 + 2 * 100 + 12 * 31
_REPLICAS = 8

_NC = 2
_NS = 16
_NW = _NC * _NS
_S_PER_W = BATCH // _NW
_CHUNK = 128
_NBLOCK = _S_PER_W // _CHUNK
_IDX_PER_W = _S_PER_W * NUM_PAIRS

_mesh = plsc.VectorSubcoreMesh(core_axis_name="c", subcore_axis_name="s")


@functools.partial(
    pl.kernel,
    out_type=jax.ShapeDtypeStruct((BATCH, NUM_PAIRS * PAIR_DIM), jnp.float32),
    mesh=_mesh,
    scratch_types=[
        pltpu.VMEM((_IDX_PER_W,), jnp.int32),
        pltpu.VMEM((_CHUNK, NUM_PAIRS * PAIR_DIM), jnp.float32),
        pltpu.VMEM((_CHUNK, NUM_PAIRS * PAIR_DIM), jnp.float32),
        pltpu.SemaphoreType.DMA,
        pltpu.SemaphoreType.DMA,
    ],
)
def _embed_concat(idx_hbm, table_hbm, out, idx_v, buf_a, buf_b, gsem, osem):
    bufs = (buf_a, buf_b)
    sid = lax.axis_index("s")
    wid = sid * _NC + lax.axis_index("c")

    pltpu.sync_copy(idx_hbm.at[pl.ds(wid * _IDX_PER_W, _IDX_PER_W)], idx_v)

    out_writes = []
    for b in range(_NBLOCK):
        buf = bufs[b % 2]
        if b >= 2:
            out_writes[b - 2].wait()
        gathers = [
            pltpu.async_copy(
                table_hbm.at[idx_v.at[pl.ds((b * NUM_PAIRS + c) * _CHUNK,
                                           _CHUNK)]],
                buf.at[:, pl.ds(c * PAIR_DIM, PAIR_DIM)],
                gsem,
            )
            for c in range(NUM_PAIRS)
        ]
        for g in gathers:
            g.wait()
        out_writes.append(
            pltpu.async_copy(
                buf,
                out.at[pl.ds(wid * _S_PER_W + b * _CHUNK, _CHUNK)],
                osem,
            )
        )
    out_writes[_NBLOCK - 2].wait()
    out_writes[_NBLOCK - 1].wait()


def _pair_table(wa, wb):
    va, vb = wa.shape[0], wb.shape[0]
    return jnp.concatenate(
        [jnp.broadcast_to(wa[:, None, :], (va, vb, NUM_DIM)),
         jnp.broadcast_to(wb[None, :, :], (va, vb, NUM_DIM))],
        axis=-1,
    ).reshape(va * vb, PAIR_DIM)


def kernel(dayofweek, time, sex, age, month, day,
           W_dayofweek, W_time, W_sex, W_age, W_month, W_day):
    table = jnp.concatenate(
        [_pair_table(W_dayofweek, W_time),
         _pair_table(W_sex, W_age),
         _pair_table(W_month, W_day)],
        axis=0,
    )
    table_rep = jnp.broadcast_to(
        table[None], (_REPLICAS, TABLE_ROWS, PAIR_DIM)
    ).reshape(_REPLICAS * TABLE_ROWS, PAIR_DIM)
    p0 = dayofweek.astype(jnp.int32) * 24 + time.astype(jnp.int32)
    p1 = 168 + sex.astype(jnp.int32) * 100 + age.astype(jnp.int32)
    p2 = 368 + month.astype(jnp.int32) * 31 + day.astype(jnp.int32)
    idx = (jnp.stack([p0, p1, p2], axis=-1)
           .reshape(BATCH // _CHUNK, _CHUNK, NUM_PAIRS)
           .transpose(0, 2, 1)
           .reshape(-1))
    stream_id = jnp.arange(idx.shape[0], dtype=jnp.int32) // _CHUNK
    idx = idx + (stream_id % _REPLICAS) * TABLE_ROWS
    return _embed_concat(idx, table_rep)
